# baseline (speedup 1.0000x reference)
import jax
import jax.numpy as jnp
from jax import lax
from jax.experimental import pallas as pl
from jax.experimental.pallas import tpu as pltpu
from jax.experimental.pallas import tpu_sc as plsc

_MARK_AX = 0
_OP_SHL = 1
_OP_SHR = 2
_ALU_LO = 3
_ALU_HI = 19
_AX_CARRY_LO = 35
_OUTPUT_LO = 51
_OUTPUT_HI = 67

_NC = 2
_NS = 16
_L = 16
_NW = _NC * _NS

_B = 8
_S = 2048
_F = 96
_ROWS_PER_W = _B * _S // _NW
_GROUPS = _ROWS_PER_W // _L
_W_PER_B = _S // _ROWS_PER_W


def _sc_body(x_hbm, out_hbm, buf):
    c = lax.axis_index("c")
    s = lax.axis_index("s")
    wid = s * _NC + c
    bi = wid // _W_PER_B
    r0 = (wid % _W_PER_B) * _ROWS_PER_W

    pltpu.sync_copy(x_hbm.at[bi, pl.ds(r0, _ROWS_PER_W)], buf)
    lane_iota = lax.iota(jnp.int32, _L)

    @plsc.parallel_loop(0, _GROUPS, 1, unroll=4)
    def group(g):
        rows = lane_iota + g * _L

        def gcol(col):
            return plsc.load_gather(buf, [rows, jnp.full((_L,), col, jnp.int32)])

        def wargmax(lo):
            best = gcol(lo)
            besti = jnp.zeros((_L,), jnp.int32)
            for j in range(1, 16):
                v = gcol(lo + j)
                m = v > best
                best = jnp.where(m, v, best)
                besti = jnp.where(m, j, besti)
            return besti

        val_lo = wargmax(_ALU_LO)
        val_hi = wargmax(_ALU_HI)
        shift_amt = jnp.minimum(wargmax(_AX_CARRY_LO), 31)

        active = (gcol(_MARK_AX) >= 0.5) & (
            (gcol(_OP_SHL) > 0.5) | (gcol(_OP_SHR) > 0.5))
        is_shl = gcol(_OP_SHL) > 0.5

        value = val_lo + (val_hi << 4)
        shl_res = (value << shift_amt) & 255
        shr_res = lax.shift_right_logical(value, shift_amt)
        result = jnp.where(is_shl, shl_res, shr_res)

        two = jnp.full((_L,), 2.0, jnp.float32)
        plsc.addupdate_scatter(
            buf, [rows, (result & 15) + _OUTPUT_LO], two, mask=active)
        plsc.addupdate_scatter(
            buf, [rows, (result >> 4) + _OUTPUT_HI], two, mask=active)

    pltpu.sync_copy(buf, out_hbm.at[bi, pl.ds(r0, _ROWS_PER_W)])


def kernel(x_bd):
    b, sq, f = x_bd.shape
    mesh = plsc.VectorSubcoreMesh(
        core_axis_name="c", subcore_axis_name="s",
        num_cores=_NC, num_subcores=_NS)
    run = pl.kernel(
        _sc_body,
        out_type=jax.ShapeDtypeStruct((b, sq, f), x_bd.dtype),
        mesh=mesh,
        scratch_types=[pltpu.VMEM((_ROWS_PER_W, f), jnp.float32)],
        compiler_params=pltpu.CompilerParams(needs_layout_passes=False),
    )
    return run(x_bd)


# speedup vs baseline: 1.0285x; 1.0285x over previous
import jax
import jax.numpy as jnp
from jax import lax
from jax.experimental import pallas as pl
from jax.experimental.pallas import tpu as pltpu
from jax.experimental.pallas import tpu_sc as plsc

_MARK_AX = 0
_OP_SHL = 1
_OP_SHR = 2
_ALU_LO = 3
_ALU_HI = 19
_AX_CARRY_LO = 35
_OUTPUT_LO = 51
_OUTPUT_HI = 67

_NC = 2
_NS = 16
_L = 16
_NW = _NC * _NS

_B = 8
_S = 2048
_F = 96
_ROWS_PER_W = _B * _S // _NW
_GROUPS = _ROWS_PER_W // _L
_W_PER_B = _S // _ROWS_PER_W


def _sc_body(x_hbm, out_hbm, buf):
    c = lax.axis_index("c")
    s = lax.axis_index("s")
    wid = s * _NC + c
    bi = wid // _W_PER_B
    r0 = (wid % _W_PER_B) * _ROWS_PER_W

    pltpu.sync_copy(x_hbm.at[bi, pl.ds(r0, _ROWS_PER_W)], buf)
    lane_iota = lax.iota(jnp.int32, _L)

    @plsc.parallel_loop(0, _GROUPS, 1, unroll=2)
    def group(g):
        rows = lane_iota + g * _L

        def gcol(col):
            return plsc.load_gather(buf, [rows, jnp.full((_L,), col, jnp.int32)])

        def wargmax(lo):
            vals = [gcol(lo + j) for j in range(16)]
            idxs = [jnp.full((_L,), j, jnp.int32) for j in range(16)]
            while len(vals) > 1:
                nv, ni = [], []
                for a in range(0, len(vals), 2):
                    m = vals[a + 1] > vals[a]
                    nv.append(jnp.where(m, vals[a + 1], vals[a]))
                    ni.append(jnp.where(m, idxs[a + 1], idxs[a]))
                vals, idxs = nv, ni
            return idxs[0]

        val_lo = wargmax(_ALU_LO)
        val_hi = wargmax(_ALU_HI)
        shift_amt = jnp.minimum(wargmax(_AX_CARRY_LO), 31)

        active = (gcol(_MARK_AX) >= 0.5) & (
            (gcol(_OP_SHL) > 0.5) | (gcol(_OP_SHR) > 0.5))
        is_shl = gcol(_OP_SHL) > 0.5

        value = val_lo + (val_hi << 4)
        shl_res = (value << shift_amt) & 255
        shr_res = lax.shift_right_logical(value, shift_amt)
        result = jnp.where(is_shl, shl_res, shr_res)

        two = jnp.full((_L,), 2.0, jnp.float32)
        plsc.addupdate_scatter(
            buf, [rows, (result & 15) + _OUTPUT_LO], two, mask=active)
        plsc.addupdate_scatter(
            buf, [rows, (result >> 4) + _OUTPUT_HI], two, mask=active)

    pltpu.sync_copy(buf, out_hbm.at[bi, pl.ds(r0, _ROWS_PER_W)])


def kernel(x_bd):
    b, sq, f = x_bd.shape
    mesh = plsc.VectorSubcoreMesh(
        core_axis_name="c", subcore_axis_name="s",
        num_cores=_NC, num_subcores=_NS)
    run = pl.kernel(
        _sc_body,
        out_type=jax.ShapeDtypeStruct((b, sq, f), x_bd.dtype),
        mesh=mesh,
        scratch_types=[pltpu.VMEM((_ROWS_PER_W, f), jnp.float32)],
        compiler_params=pltpu.CompilerParams(needs_layout_passes=False),
    )
    return run(x_bd)


# speedup vs baseline: 1.1533x; 1.1214x over previous
import jax
import jax.numpy as jnp
from jax import lax
from jax.experimental import pallas as pl
from jax.experimental.pallas import tpu as pltpu

_MARK_AX = 0
_OP_SHL = 1
_OP_SHR = 2
_ALU_LO = 3
_ALU_HI = 19
_AX_CARRY_LO = 35
_OUTPUT_LO = 51
_OUTPUT_HI = 67

_ROWS_PER_BLOCK = 512


def _body(x_ref, o_ref):
    x = x_ref[...]
    r, f = x.shape
    i = lax.broadcasted_iota(jnp.int32, (r, f), 1)
    neg = jnp.float32(-jnp.inf)

    def window_argmax(lo):
        m = jnp.where((i >= lo) & (i < lo + 16), x, neg)
        return (jnp.argmax(m, axis=1).astype(jnp.int32) - lo)[:, None]

    val_lo = window_argmax(_ALU_LO)
    val_hi = window_argmax(_ALU_HI)
    shift_amt = jnp.minimum(window_argmax(_AX_CARRY_LO), 31)

    mark = x[:, _MARK_AX:_MARK_AX + 1] >= 0.5
    is_shl = x[:, _OP_SHL:_OP_SHL + 1] > 0.5
    is_shr = x[:, _OP_SHR:_OP_SHR + 1] > 0.5
    active = mark & (is_shl | is_shr)

    value = val_lo + (val_hi << 4)
    shl_res = (value << shift_amt) & 255
    shr_res = value >> shift_amt
    result = jnp.where(is_shl, shl_res, shr_res)

    hit = (i == (result & 15) + _OUTPUT_LO) | (i == (result >> 4) + _OUTPUT_HI)
    add = jnp.where(active & hit, jnp.float32(2.0), jnp.float32(0.0))
    o_ref[...] = x + add


def kernel(x_bd):
    b, sq, f = x_bd.shape
    nblk = sq // _ROWS_PER_BLOCK
    out = pl.pallas_call(
        _body,
        grid=(b, nblk),
        in_specs=[pl.BlockSpec((None, _ROWS_PER_BLOCK, f),
                               lambda bi, ri: (bi, ri, 0))],
        out_specs=pl.BlockSpec((None, _ROWS_PER_BLOCK, f),
                               lambda bi, ri: (bi, ri, 0)),
        out_shape=jax.ShapeDtypeStruct((b, sq, f), x_bd.dtype),
    )(x_bd)
    return out


# speedup vs baseline: 5.5050x; 4.7731x over previous
import jax
import jax.numpy as jnp
from jax import lax
from jax.experimental import pallas as pl
from jax.experimental.pallas import tpu as pltpu

_MARK_AX = 0
_OP_SHL = 1
_OP_SHR = 2
_ALU_LO = 3
_ALU_HI = 19
_AX_CARRY_LO = 35
_OUTPUT_LO = 51
_OUTPUT_HI = 67

_LANES_PER_BLOCK = 2048


def _body(x_ref, o_ref):
    x = x_ref[...]
    f, l = x.shape

    w = x[_ALU_LO:_ALU_LO + 48, :]
    fi = lax.broadcasted_iota(jnp.int32, (48, l), 0)
    v = w
    idx = fi
    for s in (1, 2, 4, 8):
        vs = pltpu.roll(v, 48 - s, 0)
        ixs = pltpu.roll(idx, 48 - s, 0)
        m = vs > v
        v = jnp.where(m, vs, v)
        idx = jnp.where(m, ixs, idx)
    rel = idx - fi

    val_lo = rel[0:1, :]
    val_hi = rel[16:17, :]
    shift_amt = jnp.minimum(rel[32:33, :], 31)

    mark = x[_MARK_AX:_MARK_AX + 1, :] >= 0.5
    is_shl = x[_OP_SHL:_OP_SHL + 1, :] > 0.5
    is_shr = x[_OP_SHR:_OP_SHR + 1, :] > 0.5
    active = mark & (is_shl | is_shr)

    value = val_lo + (val_hi << 4)
    shl_res = (value << shift_amt) & 255
    shr_res = value >> shift_amt
    result = jnp.where(is_shl, shl_res, shr_res)
    res_lo = (result & 15) + _OUTPUT_LO
    res_hi = (result >> 4) + _OUTPUT_HI

    oi = lax.broadcasted_iota(jnp.int32, (32, l), 0) + _OUTPUT_LO
    hit = (oi == res_lo) | (oi == res_hi)
    add = jnp.where(active & hit, jnp.float32(2.0), jnp.float32(0.0))

    o_ref[...] = x
    o_ref[_OUTPUT_LO:_OUTPUT_LO + 32, :] = x[_OUTPUT_LO:_OUTPUT_LO + 32, :] + add


def kernel(x_bd):
    b, sq, f = x_bd.shape
    xt = jnp.transpose(x_bd, (0, 2, 1))
    nblk = sq // _LANES_PER_BLOCK
    out_t = pl.pallas_call(
        _body,
        grid=(b, nblk),
        in_specs=[pl.BlockSpec((None, f, _LANES_PER_BLOCK),
                               lambda bi, ri: (bi, 0, ri))],
        out_specs=pl.BlockSpec((None, f, _LANES_PER_BLOCK),
                               lambda bi, ri: (bi, 0, ri)),
        out_shape=jax.ShapeDtypeStruct((b, f, sq), x_bd.dtype),
    )(xt)
    return jnp.transpose(out_t, (0, 2, 1))


# speedup vs baseline: 6.7908x; 1.2336x over previous
import jax
import jax.numpy as jnp
from jax import lax
from jax.experimental import pallas as pl
from jax.experimental.pallas import tpu as pltpu

_MARK_AX = 0
_OP_SHL = 1
_OP_SHR = 2
_ALU_LO = 3
_ALU_HI = 19
_AX_CARRY_LO = 35
_OUTPUT_LO = 51
_OUTPUT_HI = 67

_BATCH_PER_BLOCK = 2


def _body(x_ref, o_ref):
    x = x_ref[...]
    b2, f, l = x.shape

    w = x[:, _ALU_LO:_ALU_LO + 48, :]
    fi = lax.broadcasted_iota(jnp.int32, (b2, 48, l), 1)
    v = w
    idx = fi
    for s in (1, 2, 4, 8):
        vs = pltpu.roll(v, 48 - s, 1)
        ixs = pltpu.roll(idx, 48 - s, 1)
        m = vs > v
        v = jnp.where(m, vs, v)
        idx = jnp.where(m, ixs, idx)
    rel = idx - fi

    val_lo = rel[:, 0:1, :]
    val_hi = rel[:, 16:17, :]
    shift_amt = jnp.minimum(rel[:, 32:33, :], 31)

    mark = x[:, _MARK_AX:_MARK_AX + 1, :] >= 0.5
    is_shl = x[:, _OP_SHL:_OP_SHL + 1, :] > 0.5
    is_shr = x[:, _OP_SHR:_OP_SHR + 1, :] > 0.5
    active = mark & (is_shl | is_shr)

    value = val_lo + (val_hi << 4)
    shl_res = (value << shift_amt) & 255
    shr_res = value >> shift_amt
    result = jnp.where(is_shl, shl_res, shr_res)
    res_lo = (result & 15) + _OUTPUT_LO
    res_hi = (result >> 4) + _OUTPUT_HI

    oi = lax.broadcasted_iota(jnp.int32, (b2, 32, l), 1) + _OUTPUT_LO
    hit = (oi == res_lo) | (oi == res_hi)
    add = jnp.where(active & hit, jnp.float32(2.0), jnp.float32(0.0))

    o_ref[...] = x
    o_ref[:, _OUTPUT_LO:_OUTPUT_LO + 32, :] = (
        x[:, _OUTPUT_LO:_OUTPUT_LO + 32, :] + add)


def kernel(x_bd):
    b, sq, f = x_bd.shape
    xt = jnp.transpose(x_bd, (0, 2, 1))
    out_t = pl.pallas_call(
        _body,
        grid=(b // _BATCH_PER_BLOCK,),
        in_specs=[pl.BlockSpec((_BATCH_PER_BLOCK, f, sq),
                               lambda bi: (bi, 0, 0))],
        out_specs=pl.BlockSpec((_BATCH_PER_BLOCK, f, sq),
                               lambda bi: (bi, 0, 0)),
        out_shape=jax.ShapeDtypeStruct((b, f, sq), x_bd.dtype),
    )(xt)
    return jnp.transpose(out_t, (0, 2, 1))


# speedup vs baseline: 7.1379x; 1.0511x over previous
import jax
import jax.numpy as jnp
from jax import lax
from jax.experimental import pallas as pl
from jax.experimental.pallas import tpu as pltpu

_MARK_AX = 0
_OP_SHL = 1
_OP_SHR = 2
_ALU_LO = 3
_ALU_HI = 19
_AX_CARRY_LO = 35
_OUTPUT_LO = 51
_OUTPUT_HI = 67

_BATCH_PER_BLOCK = 4


def _body(x_ref, o_ref):
    x = x_ref[...]
    b2, f, l = x.shape

    w = x[:, _ALU_LO:_ALU_LO + 48, :]
    fi = lax.broadcasted_iota(jnp.int32, (b2, 48, l), 1)
    v = w
    idx = fi
    for s in (1, 2, 4, 8):
        vs = pltpu.roll(v, 48 - s, 1)
        ixs = pltpu.roll(idx, 48 - s, 1)
        m = vs > v
        v = jnp.where(m, vs, v)
        idx = jnp.where(m, ixs, idx)
    rel = idx - fi

    val_lo = rel[:, 0:1, :]
    val_hi = rel[:, 16:17, :]
    shift_amt = jnp.minimum(rel[:, 32:33, :], 31)

    mark = x[:, _MARK_AX:_MARK_AX + 1, :] >= 0.5
    is_shl = x[:, _OP_SHL:_OP_SHL + 1, :] > 0.5
    is_shr = x[:, _OP_SHR:_OP_SHR + 1, :] > 0.5
    active = mark & (is_shl | is_shr)

    value = val_lo + (val_hi << 4)
    shl_res = (value << shift_amt) & 255
    shr_res = value >> shift_amt
    result = jnp.where(is_shl, shl_res, shr_res)
    res_lo = (result & 15) + _OUTPUT_LO
    res_hi = (result >> 4) + _OUTPUT_HI

    oi = lax.broadcasted_iota(jnp.int32, (b2, 32, l), 1) + _OUTPUT_LO
    hit = (oi == res_lo) | (oi == res_hi)
    add = jnp.where(active & hit, jnp.float32(2.0), jnp.float32(0.0))

    o_ref[...] = x
    o_ref[:, _OUTPUT_LO:_OUTPUT_LO + 32, :] = (
        x[:, _OUTPUT_LO:_OUTPUT_LO + 32, :] + add)


def kernel(x_bd):
    b, sq, f = x_bd.shape
    xt = jnp.transpose(x_bd, (0, 2, 1))
    out_t = pl.pallas_call(
        _body,
        grid=(b // _BATCH_PER_BLOCK,),
        in_specs=[pl.BlockSpec((_BATCH_PER_BLOCK, f, sq),
                               lambda bi: (bi, 0, 0))],
        out_specs=pl.BlockSpec((_BATCH_PER_BLOCK, f, sq),
                               lambda bi: (bi, 0, 0)),
        out_shape=jax.ShapeDtypeStruct((b, f, sq), x_bd.dtype),
    )(xt)
    return jnp.transpose(out_t, (0, 2, 1))
